# baseline (speedup 1.0000x reference)
import functools

import jax
import jax.numpy as jnp
from jax import lax
from jax.experimental import pallas as pl
from jax.experimental.pallas import tpu as pltpu
from jax.experimental.pallas import tpu_sc as plsc

BATCH = 16384
DIM = 32
LANES = 16

_info = plsc.get_sparse_core_info()
NC = _info.num_cores
NS = _info.num_subcores
NW = NC * NS
NUM_ROWS = 1000000
CHUNK = 768
POS_BITS = 14
POS_MASK = (1 << POS_BITS) - 1

_BOUNDS = [(w * (NUM_ROWS // NW)) // CHUNK * CHUNK for w in range(NW)]
_BOUNDS += [NUM_ROWS]
_MAXNCH = max((_BOUNDS[w + 1] - _BOUNDS[w]) // CHUNK for w in range(NW))

_mesh = plsc.VectorSubcoreMesh(core_axis_name="c", subcore_axis_name="s")


@functools.partial(
    pl.kernel,
    mesh=_mesh,
    compiler_params=pltpu.CompilerParams(
        needs_layout_passes=False, use_tc_tiling_on_sc=True),
    out_type=(jax.ShapeDtypeStruct((BATCH, DIM), jnp.float32),
              jax.ShapeDtypeStruct((BATCH, DIM), jnp.float32)),
    scratch_types=[
        pltpu.VMEM((BATCH,), jnp.int32),
        pltpu.VMEM((BATCH,), jnp.int32),
        pltpu.VMEM((DIM, CHUNK), jnp.float32),
        pltpu.VMEM((DIM, CHUNK), jnp.float32),
        pltpu.VMEM((DIM, 64), jnp.float32),
        pltpu.VMEM((2 * LANES,), jnp.int32),
        pltpu.VMEM((2 * LANES,), jnp.int32),
        pltpu.VMEM((8, LANES, DIM), jnp.float32),
        pltpu.VMEM((8, LANES, DIM), jnp.float32),
        pltpu.SemaphoreType.DMA,
        pltpu.SemaphoreType.DMA,
        pltpu.SemaphoreType.DMA,
        pltpu.SemaphoreType.DMA,
    ],
)
def _gather_stream(table_t_hbm, tail_t_hbm, ids1_hbm, ids2_hbm, g1_hbm,
                   g2_hbm,
                   lst1, lst2, buf0, buf1, tbuf, pend1, pend2, stag1, stag2,
                   semb0, semb1, seme1, seme2):
    wid = lax.axis_index("s") * NC + lax.axis_index("c")
    lo = (wid * (NUM_ROWS // NW)) // CHUNK * CHUNK
    hi_next = ((wid + 1) * (NUM_ROWS // NW)) // CHUNK * CHUNK
    hi = jnp.where(wid == NW - 1, NUM_ROWS, hi_next)
    nch = (hi_next - lo) // CHUNK

    lanesv = lax.iota(jnp.int32, LANES)

    pltpu.sync_copy(ids1_hbm, lst1)
    pltpu.sync_copy(ids2_hbm, lst2)

    def scan_body(i, cnts):
        c1, c2 = cnts
        pos = i * LANES + lanesv
        idv1 = lst1[pl.ds(i * LANES, LANES)]
        idv2 = lst2[pl.ds(i * LANES, LANES)]
        m1 = (idv1 >= lo) & (idv1 < hi)
        m2 = (idv2 >= lo) & (idv2 < hi)
        e1 = ((idv1 - lo) << POS_BITS) | pos
        e2 = ((idv2 - lo) << POS_BITS) | pos
        plsc.store_compressed(lst1.at[pl.ds(c1, LANES)], e1, mask=m1)
        plsc.store_compressed(lst2.at[pl.ds(c2, LANES)], e2, mask=m2)
        c1 = c1 + plsc.all_reduce_population_count(m1)[0]
        c2 = c2 + plsc.all_reduce_population_count(m2)[0]
        return c1, c2

    cnt1, cnt2 = lax.fori_loop(0, BATCH // LANES, scan_body,
                               (jnp.int32(0), jnp.int32(0)))

    def fire(q, buf, semb):
        off = pl.multiple_of(lo + q * CHUNK, 128)
        pltpu.async_copy(table_t_hbm.at[:, pl.ds(off, CHUNK)], buf, semb)

    def bwait(buf, semb):
        pltpu.make_async_copy(
            table_t_hbm.at[:, pl.ds(0, CHUNK)], buf, semb).wait()

    def drain_group(stag, g_hbm, seme, n):
        def drain(s, carry):
            pltpu.make_async_copy(g_hbm.at[pl.ds(0, LANES)], stag.at[0],
                                  seme).wait()
            return carry
        lax.fori_loop(0, n, drain, 0)

    def emit(buf, k, qrel, pend, stag, g_hbm, seme, ecnt):
        @pl.when((ecnt > 0) & (ecnt % 8 == 0))
        def _():
            drain_group(stag, g_hbm, seme, jnp.int32(8))

        slot = stag.at[ecnt % 8]
        ev = pend[pl.ds(0, LANES)]
        emask = lanesv < k
        colv = jnp.where(emask, (ev >> POS_BITS) - qrel, 0)
        posv = ev & POS_MASK
        for c in range(DIM):
            cc = jnp.full((LANES,), c, jnp.int32)
            vals = plsc.load_gather(buf, [cc, colv], mask=emask)
            ccr = (cc + posv) & (DIM - 1)
            plsc.store_scatter(slot, [lanesv, ccr], vals, mask=emask)
        p0 = posv[0]
        for s in range(LANES):
            sidx = jnp.where(s < k, s, 0)
            ps = jnp.where(s < k, posv[s], p0)
            pltpu.async_copy(slot.at[pl.ds(sidx, 1)],
                             g_hbm.at[pl.ds(ps, 1)], seme)

    def process_side(buf, width, qrel, cnt, lst, pend, stag, g_hbm, seme,
                     ecnt0):
        nvec = (cnt + LANES - 1) // LANES

        def body(j, carry):
            pcnt, ecnt = carry
            ev = lst[pl.ds(j * LANES, LANES)]
            rel = (ev >> POS_BITS) - qrel
            vmask = (j * LANES + lanesv) < cnt
            m = vmask & (rel >= 0) & (rel < width)
            plsc.store_compressed(pend.at[pl.ds(pcnt, LANES)], ev, mask=m)
            pcnt = pcnt + plsc.all_reduce_population_count(m)[0]
            full = pcnt >= LANES

            @pl.when(full)
            def _():
                emit(buf, jnp.int32(LANES), qrel, pend, stag, g_hbm, seme,
                     ecnt)
                pend[pl.ds(0, LANES)] = pend[pl.ds(LANES, LANES)]

            return (jnp.where(full, pcnt - LANES, pcnt),
                    jnp.where(full, ecnt + 1, ecnt))

        pcnt, ecnt = lax.fori_loop(0, nvec, body, (jnp.int32(0), ecnt0))

        @pl.when(pcnt > 0)
        def _():
            emit(buf, pcnt, qrel, pend, stag, g_hbm, seme, ecnt)

        return jnp.where(pcnt > 0, ecnt + 1, ecnt)

    def process(buf, width, qrel, ecnts):
        e1 = process_side(buf, width, qrel, cnt1, lst1, pend1, stag1,
                          g1_hbm, seme1, ecnts[0])
        e2 = process_side(buf, width, qrel, cnt2, lst2, pend2, stag2,
                          g2_hbm, seme2, ecnts[1])
        return (e1, e2)

    fire(0, buf0, semb0)

    @pl.when(nch > 1)
    def _():
        fire(1, buf1, semb1)

    def do_chunk(q, buf, semb, ecnts):
        def hit(ec):
            bwait(buf, semb)
            ec = process(buf, CHUNK, q * CHUNK, ec)

            @pl.when(q + 2 < nch)
            def _():
                fire(q + 2, buf, semb)

            return ec

        return lax.cond(q < nch, hit, lambda ec: ec, ecnts)

    def chunk_pair(i, ecnts):
        ecnts = do_chunk(2 * i, buf0, semb0, ecnts)
        ecnts = do_chunk(2 * i + 1, buf1, semb1, ecnts)
        return ecnts

    ecnts = lax.fori_loop(0, (_MAXNCH + 1) // 2, chunk_pair,
                          (jnp.int32(0), jnp.int32(0)))

    def tail31(ec):
        pltpu.sync_copy(tail_t_hbm, tbuf)
        return process(tbuf, 64, jnp.int32(NUM_ROWS - 64) - lo, ec)

    e1, e2 = lax.cond(wid == NW - 1, tail31, lambda ec: ec, ecnts)

    nd1 = jnp.where(e1 > 0, e1 - 8 * ((e1 - 1) // 8), 0)
    nd2 = jnp.where(e2 > 0, e2 - 8 * ((e2 - 1) // 8), 0)
    drain_group(stag1, g1_hbm, seme1, nd1)
    drain_group(stag2, g2_hbm, seme2, nd2)


def _dot_body(g1_ref, g2_ref, out_ref):
    out_ref[...] = jnp.sum(g1_ref[...] * g2_ref[...], axis=1, keepdims=True)


_TC_BLOCK = 2048


@jax.jit
def _row_dot(g1, g2):
    return pl.pallas_call(
        _dot_body,
        grid=(BATCH // _TC_BLOCK,),
        in_specs=[
            pl.BlockSpec((_TC_BLOCK, DIM), lambda i: (i, 0)),
            pl.BlockSpec((_TC_BLOCK, DIM), lambda i: (i, 0)),
        ],
        out_specs=pl.BlockSpec((_TC_BLOCK, 1), lambda i: (i, 0)),
        out_shape=jax.ShapeDtypeStruct((BATCH, 1), jnp.float32),
    )(g1, g2)


def kernel(all_gembs, ids_1, ids_2):
    g1, g2 = _gather_stream(all_gembs.T,
                            all_gembs[NUM_ROWS - 64:].T,
                            ids_1.astype(jnp.int32),
                            ids_2.astype(jnp.int32))
    return _row_dot(g1, g2)


# speedup vs baseline: 1.0425x; 1.0425x over previous
import functools

import jax
import jax.numpy as jnp
from jax import lax
from jax.experimental import pallas as pl
from jax.experimental.pallas import tpu as pltpu
from jax.experimental.pallas import tpu_sc as plsc

BATCH = 16384
DIM = 32
LANES = 16

_info = plsc.get_sparse_core_info()
NC = _info.num_cores
NS = _info.num_subcores
NW = NC * NS
NUM_ROWS = 1000000
CHUNK = 768
POS_BITS = 14
POS_MASK = (1 << POS_BITS) - 1

_BOUNDS = [(w * (NUM_ROWS // NW)) // CHUNK * CHUNK for w in range(NW)]
_BOUNDS += [NUM_ROWS]
_MAXNCH = max((_BOUNDS[w + 1] - _BOUNDS[w]) // CHUNK for w in range(NW))

_mesh = plsc.VectorSubcoreMesh(core_axis_name="c", subcore_axis_name="s")


@functools.partial(
    pl.kernel,
    mesh=_mesh,
    compiler_params=pltpu.CompilerParams(
        needs_layout_passes=False, use_tc_tiling_on_sc=True),
    out_type=(jax.ShapeDtypeStruct((BATCH, DIM), jnp.float32),
              jax.ShapeDtypeStruct((BATCH, DIM), jnp.float32)),
    scratch_types=[
        pltpu.VMEM((BATCH,), jnp.int32),
        pltpu.VMEM((BATCH,), jnp.int32),
        pltpu.VMEM((DIM, CHUNK), jnp.float32),
        pltpu.VMEM((DIM, CHUNK), jnp.float32),
        pltpu.VMEM((DIM, 64), jnp.float32),
        pltpu.VMEM((2 * LANES,), jnp.int32),
        pltpu.VMEM((2 * LANES,), jnp.int32),
        pltpu.VMEM((8, LANES, DIM), jnp.float32),
        pltpu.VMEM((8, LANES, DIM), jnp.float32),
        pltpu.SemaphoreType.DMA,
        pltpu.SemaphoreType.DMA,
        pltpu.SemaphoreType.DMA,
        pltpu.SemaphoreType.DMA,
    ],
)
def _gather_stream(table_t_hbm, tail_t_hbm, ids1_hbm, ids2_hbm, g1_hbm,
                   g2_hbm,
                   lst1, lst2, buf0, buf1, tbuf, pend1, pend2, stag1, stag2,
                   semb0, semb1, seme1, seme2):
    wid = lax.axis_index("s") * NC + lax.axis_index("c")
    lo = (wid * (NUM_ROWS // NW)) // CHUNK * CHUNK
    hi_next = ((wid + 1) * (NUM_ROWS // NW)) // CHUNK * CHUNK
    hi = jnp.where(wid == NW - 1, NUM_ROWS, hi_next)
    nch = (hi_next - lo) // CHUNK

    lanesv = lax.iota(jnp.int32, LANES)

    pltpu.sync_copy(ids1_hbm, lst1)
    pltpu.sync_copy(ids2_hbm, lst2)

    def scan_body(i, cnts):
        c1, c2 = cnts
        pos = i * LANES + lanesv
        idv1 = lst1[pl.ds(i * LANES, LANES)]
        idv2 = lst2[pl.ds(i * LANES, LANES)]
        m1 = (idv1 >= lo) & (idv1 < hi)
        m2 = (idv2 >= lo) & (idv2 < hi)
        e1 = ((idv1 - lo) << POS_BITS) | pos
        e2 = ((idv2 - lo) << POS_BITS) | pos
        plsc.store_compressed(lst1.at[pl.ds(c1, LANES)], e1, mask=m1)
        plsc.store_compressed(lst2.at[pl.ds(c2, LANES)], e2, mask=m2)
        c1 = c1 + plsc.all_reduce_population_count(m1)[0]
        c2 = c2 + plsc.all_reduce_population_count(m2)[0]
        return c1, c2

    cnt1, cnt2 = lax.fori_loop(0, BATCH // LANES, scan_body,
                               (jnp.int32(0), jnp.int32(0)))

    def fire(q, buf, semb):
        off = pl.multiple_of(lo + q * CHUNK, 128)
        pltpu.async_copy(table_t_hbm.at[:, pl.ds(off, CHUNK)], buf, semb)

    def bwait(buf, semb):
        pltpu.make_async_copy(
            table_t_hbm.at[:, pl.ds(0, CHUNK)], buf, semb).wait()

    def drain_group(stag, g_hbm, seme, n):
        def drain(s, carry):
            pltpu.make_async_copy(g_hbm.at[pl.ds(0, LANES)], stag.at[0],
                                  seme).wait()
            return carry
        lax.fori_loop(0, n, drain, 0)

    def emit(buf, k, qrel, pend, stag, g_hbm, seme, ecnt):
        @pl.when((ecnt > 0) & (ecnt % 8 == 0))
        def _():
            drain_group(stag, g_hbm, seme, jnp.int32(8))

        slot = stag.at[ecnt % 8]
        ev = pend[pl.ds(0, LANES)]
        emask = lanesv < k
        colv = jnp.where(emask, (ev >> POS_BITS) - qrel, 0)
        posv = ev & POS_MASK
        for c in range(DIM):
            cc = jnp.full((LANES,), c, jnp.int32)
            vals = plsc.load_gather(buf, [cc, colv], mask=emask)
            plsc.store_scatter(slot, [lanesv, cc], vals, mask=emask)
        p0 = posv[0]
        for s in range(LANES):
            sidx = jnp.where(s < k, s, 0)
            ps = jnp.where(s < k, posv[s], p0)
            pltpu.async_copy(slot.at[pl.ds(sidx, 1)],
                             g_hbm.at[pl.ds(ps, 1)], seme)

    def process_side(buf, width, qrel, cnt, lst, pend, stag, g_hbm, seme,
                     ecnt0):
        nvec = (cnt + LANES - 1) // LANES

        def body(j, carry):
            pcnt, ecnt = carry
            ev = lst[pl.ds(j * LANES, LANES)]
            rel = (ev >> POS_BITS) - qrel
            vmask = (j * LANES + lanesv) < cnt
            m = vmask & (rel >= 0) & (rel < width)
            plsc.store_compressed(pend.at[pl.ds(pcnt, LANES)], ev, mask=m)
            pcnt = pcnt + plsc.all_reduce_population_count(m)[0]
            full = pcnt >= LANES

            @pl.when(full)
            def _():
                emit(buf, jnp.int32(LANES), qrel, pend, stag, g_hbm, seme,
                     ecnt)
                pend[pl.ds(0, LANES)] = pend[pl.ds(LANES, LANES)]

            return (jnp.where(full, pcnt - LANES, pcnt),
                    jnp.where(full, ecnt + 1, ecnt))

        pcnt, ecnt = lax.fori_loop(0, nvec, body, (jnp.int32(0), ecnt0))

        @pl.when(pcnt > 0)
        def _():
            emit(buf, pcnt, qrel, pend, stag, g_hbm, seme, ecnt)

        return jnp.where(pcnt > 0, ecnt + 1, ecnt)

    def process(buf, width, qrel, ecnts):
        e1 = process_side(buf, width, qrel, cnt1, lst1, pend1, stag1,
                          g1_hbm, seme1, ecnts[0])
        e2 = process_side(buf, width, qrel, cnt2, lst2, pend2, stag2,
                          g2_hbm, seme2, ecnts[1])
        return (e1, e2)

    fire(0, buf0, semb0)

    @pl.when(nch > 1)
    def _():
        fire(1, buf1, semb1)

    def do_chunk(q, buf, semb, ecnts):
        def hit(ec):
            bwait(buf, semb)
            ec = process(buf, CHUNK, q * CHUNK, ec)

            @pl.when(q + 2 < nch)
            def _():
                fire(q + 2, buf, semb)

            return ec

        return lax.cond(q < nch, hit, lambda ec: ec, ecnts)

    def chunk_pair(i, ecnts):
        ecnts = do_chunk(2 * i, buf0, semb0, ecnts)
        ecnts = do_chunk(2 * i + 1, buf1, semb1, ecnts)
        return ecnts

    ecnts = lax.fori_loop(0, (_MAXNCH + 1) // 2, chunk_pair,
                          (jnp.int32(0), jnp.int32(0)))

    def tail31(ec):
        pltpu.sync_copy(tail_t_hbm, tbuf)
        return process(tbuf, 64, jnp.int32(NUM_ROWS - 64) - lo, ec)

    e1, e2 = lax.cond(wid == NW - 1, tail31, lambda ec: ec, ecnts)

    nd1 = jnp.where(e1 > 0, e1 - 8 * ((e1 - 1) // 8), 0)
    nd2 = jnp.where(e2 > 0, e2 - 8 * ((e2 - 1) // 8), 0)
    drain_group(stag1, g1_hbm, seme1, nd1)
    drain_group(stag2, g2_hbm, seme2, nd2)


def _dot_body(g1_ref, g2_ref, out_ref):
    out_ref[...] = jnp.sum(g1_ref[...] * g2_ref[...], axis=1, keepdims=True)


_TC_BLOCK = 2048


@jax.jit
def _row_dot(g1, g2):
    return pl.pallas_call(
        _dot_body,
        grid=(BATCH // _TC_BLOCK,),
        in_specs=[
            pl.BlockSpec((_TC_BLOCK, DIM), lambda i: (i, 0)),
            pl.BlockSpec((_TC_BLOCK, DIM), lambda i: (i, 0)),
        ],
        out_specs=pl.BlockSpec((_TC_BLOCK, 1), lambda i: (i, 0)),
        out_shape=jax.ShapeDtypeStruct((BATCH, 1), jnp.float32),
    )(g1, g2)


def kernel(all_gembs, ids_1, ids_2):
    g1, g2 = _gather_stream(all_gembs.T,
                            all_gembs[NUM_ROWS - 64:].T,
                            ids_1.astype(jnp.int32),
                            ids_2.astype(jnp.int32))
    return _row_dot(g1, g2)


# speedup vs baseline: 1.1503x; 1.1034x over previous
import functools

import jax
import jax.numpy as jnp
from jax import lax
from jax.experimental import pallas as pl
from jax.experimental.pallas import tpu as pltpu
from jax.experimental.pallas import tpu_sc as plsc

BATCH = 16384
DIM = 32
LANES = 16

_info = plsc.get_sparse_core_info()
NC = _info.num_cores
NS = _info.num_subcores
NW = NC * NS
NUM_ROWS = 1000000
CHUNK = 1024
POS_BITS = 14
POS_MASK = (1 << POS_BITS) - 1

_BOUNDS = [(w * (NUM_ROWS // NW)) // CHUNK * CHUNK for w in range(NW)]
_BOUNDS += [NUM_ROWS]
_MAXNCH = max((_BOUNDS[w + 1] - _BOUNDS[w]) // CHUNK for w in range(NW))

_mesh = plsc.VectorSubcoreMesh(core_axis_name="c", subcore_axis_name="s")


@functools.partial(
    pl.kernel,
    mesh=_mesh,
    compiler_params=pltpu.CompilerParams(
        needs_layout_passes=False, use_tc_tiling_on_sc=True),
    out_type=(jax.ShapeDtypeStruct((BATCH, DIM), jnp.float32),
              jax.ShapeDtypeStruct((BATCH, DIM), jnp.float32)),
    scratch_types=[
        pltpu.VMEM((BATCH,), jnp.int32),
        pltpu.VMEM((BATCH,), jnp.int32),
        pltpu.VMEM((DIM, CHUNK), jnp.float32),
        pltpu.VMEM((DIM, CHUNK), jnp.float32),
        pltpu.VMEM((DIM, 64), jnp.float32),
        pltpu.VMEM((2 * LANES,), jnp.int32),
        pltpu.VMEM((2 * LANES,), jnp.int32),
        pltpu.VMEM((4, LANES, DIM), jnp.float32),
        pltpu.VMEM((4, LANES, DIM), jnp.float32),
        pltpu.SemaphoreType.DMA,
        pltpu.SemaphoreType.DMA,
        pltpu.SemaphoreType.DMA,
        pltpu.SemaphoreType.DMA,
    ],
)
def _gather_stream(table_t_hbm, tail_t_hbm, ids1_hbm, ids2_hbm, g1_hbm,
                   g2_hbm,
                   lst1, lst2, buf0, buf1, tbuf, pend1, pend2, stag1, stag2,
                   semb0, semb1, seme1, seme2):
    wid = lax.axis_index("s") * NC + lax.axis_index("c")
    lo = (wid * (NUM_ROWS // NW)) // CHUNK * CHUNK
    hi_next = ((wid + 1) * (NUM_ROWS // NW)) // CHUNK * CHUNK
    hi = jnp.where(wid == NW - 1, NUM_ROWS, hi_next)
    nch = (hi_next - lo) // CHUNK

    lanesv = lax.iota(jnp.int32, LANES)

    pltpu.sync_copy(ids1_hbm, lst1)
    pltpu.sync_copy(ids2_hbm, lst2)

    def scan_body(i, cnts):
        c1, c2 = cnts
        pos = i * LANES + lanesv
        idv1 = lst1[pl.ds(i * LANES, LANES)]
        idv2 = lst2[pl.ds(i * LANES, LANES)]
        m1 = (idv1 >= lo) & (idv1 < hi)
        m2 = (idv2 >= lo) & (idv2 < hi)
        e1 = ((idv1 - lo) << POS_BITS) | pos
        e2 = ((idv2 - lo) << POS_BITS) | pos
        plsc.store_compressed(lst1.at[pl.ds(c1, LANES)], e1, mask=m1)
        plsc.store_compressed(lst2.at[pl.ds(c2, LANES)], e2, mask=m2)
        c1 = c1 + plsc.all_reduce_population_count(m1)[0]
        c2 = c2 + plsc.all_reduce_population_count(m2)[0]
        return c1, c2

    cnt1, cnt2 = lax.fori_loop(0, BATCH // LANES, scan_body,
                               (jnp.int32(0), jnp.int32(0)))

    def fire(q, buf, semb):
        off = pl.multiple_of(lo + q * CHUNK, 128)
        pltpu.async_copy(table_t_hbm.at[:, pl.ds(off, CHUNK)], buf, semb)

    def bwait(buf, semb):
        pltpu.make_async_copy(
            table_t_hbm.at[:, pl.ds(0, CHUNK)], buf, semb).wait()

    def drain_group(stag, g_hbm, seme, n):
        def drain(s, carry):
            pltpu.make_async_copy(g_hbm.at[pl.ds(0, LANES)], stag.at[0],
                                  seme).wait()
            return carry
        lax.fori_loop(0, n, drain, 0)

    def emit(buf, k, qrel, pend, stag, g_hbm, seme, ecnt):
        @pl.when((ecnt > 0) & (ecnt % 4 == 0))
        def _():
            drain_group(stag, g_hbm, seme, jnp.int32(4))

        slot = stag.at[ecnt % 4]
        ev = pend[pl.ds(0, LANES)]
        emask = lanesv < k
        colv = jnp.where(emask, (ev >> POS_BITS) - qrel, 0)
        posv = ev & POS_MASK
        for c in range(DIM):
            cc = jnp.full((LANES,), c, jnp.int32)
            vals = plsc.load_gather(buf, [cc, colv], mask=emask)
            plsc.store_scatter(slot, [lanesv, cc], vals, mask=emask)
        p0 = posv[0]
        for s in range(LANES):
            sidx = jnp.where(s < k, s, 0)
            ps = jnp.where(s < k, posv[s], p0)
            pltpu.async_copy(slot.at[pl.ds(sidx, 1)],
                             g_hbm.at[pl.ds(ps, 1)], seme)

    def process_side(buf, width, qrel, cnt, lst, pend, stag, g_hbm, seme,
                     ecnt0):
        nvec = (cnt + LANES - 1) // LANES

        def body(j, carry):
            pcnt, ecnt = carry
            ev = lst[pl.ds(j * LANES, LANES)]
            rel = (ev >> POS_BITS) - qrel
            vmask = (j * LANES + lanesv) < cnt
            m = vmask & (rel >= 0) & (rel < width)
            plsc.store_compressed(pend.at[pl.ds(pcnt, LANES)], ev, mask=m)
            pcnt = pcnt + plsc.all_reduce_population_count(m)[0]
            full = pcnt >= LANES

            @pl.when(full)
            def _():
                emit(buf, jnp.int32(LANES), qrel, pend, stag, g_hbm, seme,
                     ecnt)
                pend[pl.ds(0, LANES)] = pend[pl.ds(LANES, LANES)]

            return (jnp.where(full, pcnt - LANES, pcnt),
                    jnp.where(full, ecnt + 1, ecnt))

        pcnt, ecnt = lax.fori_loop(0, nvec, body, (jnp.int32(0), ecnt0))

        @pl.when(pcnt > 0)
        def _():
            emit(buf, pcnt, qrel, pend, stag, g_hbm, seme, ecnt)

        return jnp.where(pcnt > 0, ecnt + 1, ecnt)

    def process(buf, width, qrel, ecnts):
        e1 = process_side(buf, width, qrel, cnt1, lst1, pend1, stag1,
                          g1_hbm, seme1, ecnts[0])
        e2 = process_side(buf, width, qrel, cnt2, lst2, pend2, stag2,
                          g2_hbm, seme2, ecnts[1])
        return (e1, e2)

    fire(0, buf0, semb0)

    @pl.when(nch > 1)
    def _():
        fire(1, buf1, semb1)

    def do_chunk(q, buf, semb, ecnts):
        def hit(ec):
            bwait(buf, semb)
            ec = process(buf, CHUNK, q * CHUNK, ec)

            @pl.when(q + 2 < nch)
            def _():
                fire(q + 2, buf, semb)

            return ec

        return lax.cond(q < nch, hit, lambda ec: ec, ecnts)

    def chunk_pair(i, ecnts):
        ecnts = do_chunk(2 * i, buf0, semb0, ecnts)
        ecnts = do_chunk(2 * i + 1, buf1, semb1, ecnts)
        return ecnts

    ecnts = lax.fori_loop(0, (_MAXNCH + 1) // 2, chunk_pair,
                          (jnp.int32(0), jnp.int32(0)))

    def tail31(ec):
        off = pl.multiple_of(hi_next, 128)
        pltpu.async_copy(
            table_t_hbm.at[:, pl.ds(off, 512)], buf0.at[:, pl.ds(0, 512)],
            semb0)
        pltpu.make_async_copy(
            table_t_hbm.at[:, pl.ds(off, 512)], buf0.at[:, pl.ds(0, 512)],
            semb0).wait()
        ec = process(buf0, 512, hi_next - lo, ec)

        pltpu.sync_copy(tail_t_hbm, tbuf)
        return process(tbuf, 64, jnp.int32(NUM_ROWS - 64) - lo, ec)

    e1, e2 = lax.cond(wid == NW - 1, tail31, lambda ec: ec, ecnts)

    nd1 = jnp.where(e1 > 0, e1 - 4 * ((e1 - 1) // 4), 0)
    nd2 = jnp.where(e2 > 0, e2 - 4 * ((e2 - 1) // 4), 0)
    drain_group(stag1, g1_hbm, seme1, nd1)
    drain_group(stag2, g2_hbm, seme2, nd2)


def _dot_body(g1_ref, g2_ref, out_ref):
    out_ref[...] = jnp.sum(g1_ref[...] * g2_ref[...], axis=1, keepdims=True)


_TC_BLOCK = 2048


@jax.jit
def _row_dot(g1, g2):
    return pl.pallas_call(
        _dot_body,
        grid=(BATCH // _TC_BLOCK,),
        in_specs=[
            pl.BlockSpec((_TC_BLOCK, DIM), lambda i: (i, 0)),
            pl.BlockSpec((_TC_BLOCK, DIM), lambda i: (i, 0)),
        ],
        out_specs=pl.BlockSpec((_TC_BLOCK, 1), lambda i: (i, 0)),
        out_shape=jax.ShapeDtypeStruct((BATCH, 1), jnp.float32),
    )(g1, g2)


def kernel(all_gembs, ids_1, ids_2):
    g1, g2 = _gather_stream(all_gembs.T,
                            all_gembs[NUM_ROWS - 64:].T,
                            ids_1.astype(jnp.int32),
                            ids_2.astype(jnp.int32))
    return _row_dot(g1, g2)
